# baseline (speedup 1.0000x reference)
import functools

import jax
import jax.numpy as jnp
from jax import lax
from jax.experimental import pallas as pl
from jax.experimental.pallas import tpu as pltpu
from jax.experimental.pallas import tpu_sc as plsc

NC = 2
NS = 16
NW = NC * NS
L = 16
CHUNK = 128
G = 8
DAUG = 144


def _sc_aggregate(n_pad, n_chunks):
    rows_per_tile = n_pad // NS
    n_groups = n_chunks // G

    mesh = plsc.VectorSubcoreMesh(core_axis_name="c", subcore_axis_name="s",
                                  num_cores=NC, num_subcores=NS)

    @functools.partial(
        pl.kernel,
        out_type=jax.ShapeDtypeStruct((NC, n_pad, DAUG), jnp.float32),
        mesh=mesh,
        scratch_types=(
            pltpu.VMEM((G, CHUNK), jnp.int32),
            pltpu.VMEM((G, CHUNK), jnp.int32),
            [pltpu.VMEM((CHUNK, DAUG), jnp.float32) for _ in range(2)],
            pltpu.VMEM_SHARED((n_pad, DAUG), jnp.float32),
            [pltpu.SemaphoreType.DMA for _ in range(2)],
            pltpu.SemaphoreType.DMA,
        ),
        compiler_params=pltpu.CompilerParams(use_tc_tiling_on_sc=False),
    )
    def body(src_hbm, dst_hbm, x_hbm, agg_out,
             srcb, dstb, rows, agg_sh, gsem, zsem):
        c = lax.axis_index("c")
        s = lax.axis_index("s")
        wid = c * NS + s
        r0 = s * rows_per_tile
        zeros16 = jnp.zeros((L,), jnp.float32)

        def zero_row(i, _):
            for k in range(DAUG // L):
                rows[0][i, pl.ds(k * L, L)] = zeros16
            return 0
        lax.fori_loop(0, CHUNK, zero_row, 0)

        descs = []
        nfull = rows_per_tile // CHUNK
        rem = rows_per_tile - nfull * CHUNK
        for k in range(nfull):
            descs.append(pltpu.async_copy(
                rows[0], agg_sh.at[pl.ds(r0 + k * CHUNK, CHUNK)], zsem))
        if rem:
            descs.append(pltpu.async_copy(
                rows[0].at[pl.ds(0, rem)],
                agg_sh.at[pl.ds(r0 + nfull * CHUNK, rem)], zsem))
        for desc in descs:
            desc.wait()

        plsc.subcore_barrier()

        pltpu.sync_copy(src_hbm.at[wid, pl.ds(0, G)], srcb)
        pltpu.sync_copy(dst_hbm.at[wid, pl.ds(0, G)], dstb)
        pltpu.async_copy(x_hbm.at[srcb.at[0]], rows[0], gsem[0])

        def group(g, _):
            for b in range(G):
                b2 = b & 1
                pltpu.make_async_copy(
                    x_hbm.at[srcb.at[b]], rows[b2], gsem[b2]).wait()
                if b < G - 1:
                    pltpu.async_copy(x_hbm.at[srcb.at[b + 1]],
                                     rows[1 - b2], gsem[1 - b2])
                    pltpu.sync_copy(rows[b2], agg_sh.at[dstb.at[b]], add=True)
                else:
                    pltpu.sync_copy(rows[b2], agg_sh.at[dstb.at[b]], add=True)

                    @pl.when(g < n_groups - 1)
                    def _():
                        pltpu.sync_copy(
                            src_hbm.at[wid, pl.ds((g + 1) * G, G)], srcb)
                        pltpu.sync_copy(
                            dst_hbm.at[wid, pl.ds((g + 1) * G, G)], dstb)
                        pltpu.async_copy(x_hbm.at[srcb.at[0]],
                                         rows[1 - b2], gsem[1 - b2])
            return 0
        lax.fori_loop(0, n_groups, group, 0)

        plsc.subcore_barrier()

        pltpu.sync_copy(agg_sh.at[pl.ds(r0, rows_per_tile)],
                        agg_out.at[c, pl.ds(r0, rows_per_tile)])

    return body


def _tc_combine(x, a0, a1, w_self, w_neigh, b2, blk):
    n, d = x.shape

    def body(x_ref, a0_ref, a1_ref, ws_ref, wn_ref, b_ref, o_ref):
        aug = a0_ref[...] + a1_ref[...]
        agg = aug[:, :d]
        degsum = jnp.maximum(aug[:, d], 1.0)
        h_neigh = agg / degsum[:, None]
        o_ref[...] = (
            jnp.dot(x_ref[...], ws_ref[...], preferred_element_type=jnp.float32)
            + jnp.dot(h_neigh, wn_ref[...], preferred_element_type=jnp.float32)
            + b_ref[...]
        )

    grid = (n // blk,)
    return pl.pallas_call(
        body,
        grid=grid,
        in_specs=[
            pl.BlockSpec((blk, d), lambda i: (i, 0)),
            pl.BlockSpec((blk, DAUG), lambda i: (i, 0)),
            pl.BlockSpec((blk, DAUG), lambda i: (i, 0)),
            pl.BlockSpec((d, d), lambda i: (0, 0)),
            pl.BlockSpec((d, d), lambda i: (0, 0)),
            pl.BlockSpec((1, d), lambda i: (0, 0)),
        ],
        out_specs=pl.BlockSpec((blk, d), lambda i: (i, 0)),
        out_shape=jax.ShapeDtypeStruct((n, d), jnp.float32),
    )(x, a0, a1, w_self, w_neigh, b2)


def kernel(inputs, edge_index, layer_id, n_layers, W_self, W_neigh, b):
    n, d = inputs.shape
    e = edge_index.shape[1]

    per_step = NW * CHUNK
    n_chunks = -(-e // (per_step * G)) * G
    e_pad = n_chunks * per_step
    n_pad = -(-(n + 1) // 128) * 128

    src = edge_index[0]
    dst = edge_index[1]
    pad = e_pad - e
    if pad:
        src = jnp.concatenate([src, jnp.zeros((pad,), jnp.int32)])
        dst = jnp.concatenate([dst, jnp.full((pad,), n, jnp.int32)])
    src_t = src.reshape(NW, n_chunks, CHUNK)
    dst_t = dst.reshape(NW, n_chunks, CHUNK)

    x_aug = jnp.concatenate(
        [inputs, jnp.ones((n, 1), jnp.float32),
         jnp.zeros((n, DAUG - d - 1), jnp.float32)], axis=1)

    agg_parts = _sc_aggregate(n_pad, n_chunks)(src_t, dst_t, x_aug)

    a0 = agg_parts[0, :n]
    a1 = agg_parts[1, :n]
    b2 = b.reshape(1, d)
    return _tc_combine(inputs, a0, a1, W_self, W_neigh, b2, blk=1000)


# speedup vs baseline: 2.8088x; 2.8088x over previous
import functools

import jax
import jax.numpy as jnp
from jax import lax
from jax.experimental import pallas as pl
from jax.experimental.pallas import tpu as pltpu
from jax.experimental.pallas import tpu_sc as plsc

NC = 2
NS = 16
NW = NC * NS
L = 16
CHUNK = 128
G = 8
DAUG = 144


def _sc_aggregate(n_pad, n_chunks):
    rows_per_tile = n_pad // NS
    n_groups = n_chunks // G

    mesh = plsc.VectorSubcoreMesh(core_axis_name="c", subcore_axis_name="s",
                                  num_cores=NC, num_subcores=NS)

    @functools.partial(
        pl.kernel,
        out_type=jax.ShapeDtypeStruct((NC, n_pad, DAUG), jnp.float32),
        mesh=mesh,
        scratch_types=(
            pltpu.VMEM((G, CHUNK), jnp.int32),
            pltpu.VMEM((G, CHUNK), jnp.int32),
            [pltpu.VMEM((CHUNK, DAUG), jnp.float32) for _ in range(2)],
            pltpu.VMEM_SHARED((n_pad, DAUG), jnp.float32),
            [pltpu.SemaphoreType.DMA for _ in range(2)],
            pltpu.SemaphoreType.DMA,
        ),
        compiler_params=pltpu.CompilerParams(use_tc_tiling_on_sc=False),
    )
    def body(src_hbm, dst_hbm, x_hbm, agg_out,
             srcb, dstb, rows, agg_sh, gsem, zsem):
        c = lax.axis_index("c")
        s = lax.axis_index("s")
        wid = c * NS + s
        r0 = s * rows_per_tile
        zeros16 = jnp.zeros((L,), jnp.float32)

        def zero_row(i, _):
            for k in range(DAUG // L):
                rows[0][i, pl.ds(k * L, L)] = zeros16
            return 0
        lax.fori_loop(0, CHUNK, zero_row, 0)

        descs = []
        nfull = rows_per_tile // CHUNK
        rem = rows_per_tile - nfull * CHUNK
        for k in range(nfull):
            descs.append(pltpu.async_copy(
                rows[0], agg_sh.at[pl.ds(r0 + k * CHUNK, CHUNK)], zsem))
        if rem:
            descs.append(pltpu.async_copy(
                rows[0].at[pl.ds(0, rem)],
                agg_sh.at[pl.ds(r0 + nfull * CHUNK, rem)], zsem))
        for desc in descs:
            desc.wait()

        plsc.subcore_barrier()

        pltpu.sync_copy(src_hbm.at[wid, pl.ds(0, G)], srcb)
        pltpu.sync_copy(dst_hbm.at[wid, pl.ds(0, G)], dstb)
        pltpu.async_copy(x_hbm.at[srcb.at[0]], rows[0], gsem[0])

        def group(g, _):
            for b in range(G):
                b2 = b & 1
                pltpu.make_async_copy(
                    x_hbm.at[srcb.at[b]], rows[b2], gsem[b2]).wait()
                if b < G - 1:
                    pltpu.async_copy(x_hbm.at[srcb.at[b + 1]],
                                     rows[1 - b2], gsem[1 - b2])
                    pltpu.sync_copy(rows[b2], agg_sh.at[dstb.at[b]], add=True)
                else:
                    pltpu.sync_copy(rows[b2], agg_sh.at[dstb.at[b]], add=True)

                    @pl.when(g < n_groups - 1)
                    def _():
                        pltpu.sync_copy(
                            src_hbm.at[wid, pl.ds((g + 1) * G, G)], srcb)
                        pltpu.sync_copy(
                            dst_hbm.at[wid, pl.ds((g + 1) * G, G)], dstb)
                        pltpu.async_copy(x_hbm.at[srcb.at[0]],
                                         rows[1 - b2], gsem[1 - b2])
            return 0
        lax.fori_loop(0, n_groups, group, 0)

        plsc.subcore_barrier()

        pltpu.sync_copy(agg_sh.at[pl.ds(r0, rows_per_tile)],
                        agg_out.at[c, pl.ds(r0, rows_per_tile)])

    return body


def _tc_combine(x, a0, a1, w_self, w_neigh, b2, blk):
    n, d = x.shape

    def body(x_ref, a0_ref, a1_ref, ws_ref, wn_ref, b_ref, o_ref):
        aug = a0_ref[...] + a1_ref[...]
        agg = aug[:, :d]
        degsum = jnp.maximum(aug[:, d], 1.0)
        h_neigh = agg / degsum[:, None]
        o_ref[...] = (
            jnp.dot(x_ref[...], ws_ref[...], preferred_element_type=jnp.float32)
            + jnp.dot(h_neigh, wn_ref[...], preferred_element_type=jnp.float32)
            + b_ref[...]
        )

    grid = (n // blk,)
    return pl.pallas_call(
        body,
        grid=grid,
        in_specs=[
            pl.BlockSpec((blk, d), lambda i: (i, 0)),
            pl.BlockSpec((blk, DAUG), lambda i: (i, 0)),
            pl.BlockSpec((blk, DAUG), lambda i: (i, 0)),
            pl.BlockSpec((d, d), lambda i: (0, 0)),
            pl.BlockSpec((d, d), lambda i: (0, 0)),
            pl.BlockSpec((1, d), lambda i: (0, 0)),
        ],
        out_specs=pl.BlockSpec((blk, d), lambda i: (i, 0)),
        out_shape=jax.ShapeDtypeStruct((n, d), jnp.float32),
    )(x, a0, a1, w_self, w_neigh, b2)


def kernel(inputs, edge_index, layer_id, n_layers, W_self, W_neigh, b):
    n, d = inputs.shape
    e = edge_index.shape[1]

    per_step = NW * CHUNK
    n_chunks = -(-e // (per_step * G)) * G
    e_pad = n_chunks * per_step
    n_pad = -(-(n + 1) // 128) * 128

    src = edge_index[0]
    dst = edge_index[1]
    pad = e_pad - e
    if pad:
        pad_ids = jnp.arange(pad, dtype=jnp.int32)
        src = jnp.concatenate([src, pad_ids % n])
        dst = jnp.concatenate([dst, n + pad_ids % (n_pad - n)])
    src_t = src.reshape(NW, n_chunks, CHUNK)
    dst_t = dst.reshape(NW, n_chunks, CHUNK)

    x_aug = jnp.concatenate(
        [inputs, jnp.ones((n, 1), jnp.float32),
         jnp.zeros((n, DAUG - d - 1), jnp.float32)], axis=1)

    agg_parts = _sc_aggregate(n_pad, n_chunks)(src_t, dst_t, x_aug)

    a0 = agg_parts[0, :n]
    a1 = agg_parts[1, :n]
    b2 = b.reshape(1, d)
    return _tc_combine(inputs, a0, a1, W_self, W_neigh, b2, blk=1000)


# speedup vs baseline: 3.4974x; 1.2452x over previous
import functools

import jax
import jax.numpy as jnp
from jax import lax
from jax.experimental import pallas as pl
from jax.experimental.pallas import tpu as pltpu
from jax.experimental.pallas import tpu_sc as plsc

NC = 2
NS = 16
NW = NC * NS
L = 16
CHUNK = 128
G = 8


def _sc_aggregate(n_pad, d, n_chunks):
    rows_per_tile = n_pad // NS
    deg_pad = -(-n_pad // (NS * L)) * (NS * L)
    deg_per_tile = deg_pad // NS
    n_groups = n_chunks // G

    mesh = plsc.VectorSubcoreMesh(core_axis_name="c", subcore_axis_name="s",
                                  num_cores=NC, num_subcores=NS)

    @functools.partial(
        pl.kernel,
        out_type=(
            jax.ShapeDtypeStruct((NC, n_pad, d), jnp.float32),
            jax.ShapeDtypeStruct((NC, deg_pad), jnp.float32),
        ),
        mesh=mesh,
        scratch_types=(
            pltpu.VMEM((G, CHUNK), jnp.int32),
            pltpu.VMEM((G, CHUNK), jnp.int32),
            [pltpu.VMEM((CHUNK, d), jnp.float32) for _ in range(2)],
            pltpu.VMEM((CHUNK,), jnp.float32),
            pltpu.VMEM((deg_per_tile,), jnp.float32),
            pltpu.VMEM_SHARED((n_pad, d), jnp.float32),
            pltpu.VMEM_SHARED((deg_pad,), jnp.float32),
            [pltpu.SemaphoreType.DMA for _ in range(2)],
            pltpu.SemaphoreType.DMA,
            pltpu.SemaphoreType.DMA,
        ),
    )
    def body(src_hbm, dst_hbm, x_hbm, agg_out, deg_out,
             srcb, dstb, rows, ones_v, zdeg_v, agg_sh, deg_sh,
             gsem, dsem, zsem):
        c = lax.axis_index("c")
        s = lax.axis_index("s")
        wid = c * NS + s
        r0 = s * rows_per_tile
        q0 = s * deg_per_tile
        zeros16 = jnp.zeros((L,), jnp.float32)
        ones16 = jnp.ones((L,), jnp.float32)

        def zero_row(i, _):
            for k in range(d // L):
                rows[0][i, pl.ds(k * L, L)] = zeros16
            return 0
        lax.fori_loop(0, CHUNK, zero_row, 0)

        descs = []
        nfull = rows_per_tile // CHUNK
        rem = rows_per_tile - nfull * CHUNK
        for k in range(nfull):
            descs.append(pltpu.async_copy(
                rows[0], agg_sh.at[pl.ds(r0 + k * CHUNK, CHUNK)], zsem))
        if rem:
            descs.append(pltpu.async_copy(
                rows[0].at[pl.ds(0, rem)],
                agg_sh.at[pl.ds(r0 + nfull * CHUNK, rem)], zsem))
        for k in range(CHUNK // L):
            ones_v[pl.ds(k * L, L)] = ones16

        def zero_deg(i, _):
            zdeg_v[pl.ds(i * L, L)] = zeros16
            return 0
        lax.fori_loop(0, deg_per_tile // L, zero_deg, 0)
        descs.append(pltpu.async_copy(
            zdeg_v, deg_sh.at[pl.ds(q0, deg_per_tile)], zsem))
        for desc in descs:
            desc.wait()

        plsc.subcore_barrier()

        pltpu.sync_copy(src_hbm.at[wid, pl.ds(0, G)], srcb)
        pltpu.sync_copy(dst_hbm.at[wid, pl.ds(0, G)], dstb)
        pltpu.async_copy(x_hbm.at[srcb.at[0]], rows[0], gsem[0])

        def group(g, _):
            for b in range(G):
                b2 = b & 1
                pltpu.make_async_copy(
                    x_hbm.at[srcb.at[b]], rows[b2], gsem[b2]).wait()
                pltpu.async_copy(ones_v, deg_sh.at[dstb.at[b]], dsem,
                                 add=True)
                if b < G - 1:
                    pltpu.async_copy(x_hbm.at[srcb.at[b + 1]],
                                     rows[1 - b2], gsem[1 - b2])
                    pltpu.sync_copy(rows[b2], agg_sh.at[dstb.at[b]], add=True)
                else:
                    pltpu.sync_copy(rows[b2], agg_sh.at[dstb.at[b]], add=True)

                    @pl.when(g < n_groups - 1)
                    def _():
                        for _b in range(G):
                            pltpu.make_async_copy(
                                ones_v, deg_sh.at[dstb.at[0]], dsem).wait()
                        pltpu.sync_copy(
                            src_hbm.at[wid, pl.ds((g + 1) * G, G)], srcb)
                        pltpu.sync_copy(
                            dst_hbm.at[wid, pl.ds((g + 1) * G, G)], dstb)
                        pltpu.async_copy(x_hbm.at[srcb.at[0]],
                                         rows[1 - b2], gsem[1 - b2])
            return 0
        lax.fori_loop(0, n_groups, group, 0)

        for _ in range(G):
            pltpu.make_async_copy(ones_v, deg_sh.at[dstb.at[0]], dsem).wait()

        plsc.subcore_barrier()

        pltpu.sync_copy(agg_sh.at[pl.ds(r0, rows_per_tile)],
                        agg_out.at[c, pl.ds(r0, rows_per_tile)])
        pltpu.sync_copy(deg_sh.at[pl.ds(q0, deg_per_tile)],
                        deg_out.at[c, pl.ds(q0, deg_per_tile)])

    return body


def _tc_combine(x, a0, a1, deg2, w_self, w_neigh, b2, blk):
    n, d = x.shape

    def body(x_ref, a0_ref, a1_ref, deg_ref, ws_ref, wn_ref, b_ref, o_ref):
        agg = a0_ref[...] + a1_ref[...]
        deg = deg_ref[...]
        degsum = jnp.maximum(deg[:, 0] + deg[:, 1], 1.0)
        h_neigh = agg / degsum[:, None]
        o_ref[...] = (
            jnp.dot(x_ref[...], ws_ref[...], preferred_element_type=jnp.float32)
            + jnp.dot(h_neigh, wn_ref[...], preferred_element_type=jnp.float32)
            + b_ref[...]
        )

    grid = (n // blk,)
    return pl.pallas_call(
        body,
        grid=grid,
        in_specs=[
            pl.BlockSpec((blk, d), lambda i: (i, 0)),
            pl.BlockSpec((blk, d), lambda i: (i, 0)),
            pl.BlockSpec((blk, d), lambda i: (i, 0)),
            pl.BlockSpec((blk, NC), lambda i: (i, 0)),
            pl.BlockSpec((d, d), lambda i: (0, 0)),
            pl.BlockSpec((d, d), lambda i: (0, 0)),
            pl.BlockSpec((1, d), lambda i: (0, 0)),
        ],
        out_specs=pl.BlockSpec((blk, d), lambda i: (i, 0)),
        out_shape=jax.ShapeDtypeStruct((n, d), jnp.float32),
    )(x, a0, a1, deg2, w_self, w_neigh, b2)


def kernel(inputs, edge_index, layer_id, n_layers, W_self, W_neigh, b):
    n, d = inputs.shape
    e = edge_index.shape[1]

    per_step = NW * CHUNK
    n_chunks = -(-e // (per_step * G)) * G
    e_pad = n_chunks * per_step
    n_pad = -(-(n + 1) // 128) * 128

    src = edge_index[0]
    dst = edge_index[1]
    pad = e_pad - e
    if pad:
        pad_ids = jnp.arange(pad, dtype=jnp.int32)
        src = jnp.concatenate([src, pad_ids % n])
        dst = jnp.concatenate([dst, n + pad_ids % (n_pad - n)])
    src_t = src.reshape(NW, n_chunks, CHUNK)
    dst_t = dst.reshape(NW, n_chunks, CHUNK)

    agg_parts, deg_parts = _sc_aggregate(n_pad, d, n_chunks)(src_t, dst_t,
                                                             inputs)

    a0 = agg_parts[0]
    a1 = agg_parts[1]
    deg2 = deg_parts.reshape(NC, -1)[:, :n].T
    b2 = b.reshape(1, d)
    return _tc_combine(inputs, a0, a1, deg2, W_self, W_neigh, b2, blk=1000)


# speedup vs baseline: 3.7163x; 1.0626x over previous
import functools

import jax
import jax.numpy as jnp
from jax import lax
from jax.experimental import pallas as pl
from jax.experimental.pallas import tpu as pltpu
from jax.experimental.pallas import tpu_sc as plsc

NC = 2
NS = 16
NW = NC * NS
L = 16
CHUNK = 128
G = 8


def _sc_aggregate(n_pad, d, n_chunks):
    rows_per_tile = n_pad // NS
    deg_pad = -(-n_pad // (NS * L)) * (NS * L)
    deg_per_tile = deg_pad // NS
    n_groups = n_chunks // G

    mesh = plsc.VectorSubcoreMesh(core_axis_name="c", subcore_axis_name="s",
                                  num_cores=NC, num_subcores=NS)

    @functools.partial(
        pl.kernel,
        out_type=(
            jax.ShapeDtypeStruct((NC, n_pad, d), jnp.float32),
            jax.ShapeDtypeStruct((NC, deg_pad), jnp.float32),
        ),
        mesh=mesh,
        scratch_types=(
            [pltpu.VMEM((G, CHUNK), jnp.int32) for _ in range(2)],
            [pltpu.VMEM((G, CHUNK), jnp.int32) for _ in range(2)],
            [pltpu.VMEM((CHUNK, d), jnp.float32) for _ in range(2)],
            pltpu.VMEM((CHUNK,), jnp.float32),
            pltpu.VMEM((deg_per_tile,), jnp.float32),
            pltpu.VMEM_SHARED((n_pad, d), jnp.float32),
            pltpu.VMEM_SHARED((deg_pad,), jnp.float32),
            [pltpu.SemaphoreType.DMA for _ in range(2)],
            pltpu.SemaphoreType.DMA,
            pltpu.SemaphoreType.DMA,
            pltpu.SemaphoreType.DMA,
        ),
    )
    def body(src_hbm, dst_hbm, x_hbm, agg_out, deg_out,
             srcb, dstb, rows, ones_v, zdeg_v, agg_sh, deg_sh,
             gsem, dsem, isem, zsem):
        c = lax.axis_index("c")
        s = lax.axis_index("s")
        wid = c * NS + s
        r0 = s * rows_per_tile
        q0 = s * deg_per_tile
        zeros16 = jnp.zeros((L,), jnp.float32)
        ones16 = jnp.ones((L,), jnp.float32)

        def zero_row(i, _):
            for k in range(d // L):
                rows[1][i, pl.ds(k * L, L)] = zeros16
            return 0
        lax.fori_loop(0, CHUNK, zero_row, 0)
        for k in range(CHUNK // L):
            ones_v[pl.ds(k * L, L)] = ones16

        def zero_deg(i, _):
            zdeg_v[pl.ds(i * L, L)] = zeros16
            return 0
        lax.fori_loop(0, deg_per_tile // L, zero_deg, 0)

        descs = []
        nfull = rows_per_tile // CHUNK
        rem = rows_per_tile - nfull * CHUNK
        for k in range(nfull):
            descs.append(pltpu.async_copy(
                rows[1], agg_sh.at[pl.ds(r0 + k * CHUNK, CHUNK)], zsem))
        if rem:
            descs.append(pltpu.async_copy(
                rows[1].at[pl.ds(0, rem)],
                agg_sh.at[pl.ds(r0 + nfull * CHUNK, rem)], zsem))
        descs.append(pltpu.async_copy(
            zdeg_v, deg_sh.at[pl.ds(q0, deg_per_tile)], zsem))

        pltpu.sync_copy(src_hbm.at[wid, pl.ds(0, G)], srcb[0])
        pltpu.sync_copy(dst_hbm.at[wid, pl.ds(0, G)], dstb[0])
        pltpu.async_copy(x_hbm.at[srcb[0].at[0]], rows[0], gsem[0])

        for desc in descs:
            desc.wait()
        plsc.subcore_barrier()

        def gpair(g2, _):
            for p in (0, 1):
                g = g2 * 2 + p

                if p == 0:
                    @pl.when(g2 > 0)
                    def _():
                        for _b in range(G):
                            pltpu.make_async_copy(
                                ones_v, deg_sh.at[dstb[1].at[0]], dsem).wait()

                    @pl.when(g < n_groups - 1)
                    def _():
                        pltpu.async_copy(
                            src_hbm.at[wid, pl.ds((g + 1) * G, G)],
                            srcb[1], isem)
                        pltpu.async_copy(
                            dst_hbm.at[wid, pl.ds((g + 1) * G, G)],
                            dstb[1], isem)
                else:
                    @pl.when(g < n_groups - 1)
                    def _():
                        for _b in range(G):
                            pltpu.make_async_copy(
                                ones_v, deg_sh.at[dstb[0].at[0]], dsem).wait()
                        pltpu.async_copy(
                            src_hbm.at[wid, pl.ds((g + 1) * G, G)],
                            srcb[0], isem)
                        pltpu.async_copy(
                            dst_hbm.at[wid, pl.ds((g + 1) * G, G)],
                            dstb[0], isem)

                for b in range(G):
                    b2 = b & 1
                    pltpu.make_async_copy(
                        x_hbm.at[srcb[p].at[b]], rows[b2], gsem[b2]).wait()
                    pltpu.async_copy(ones_v, deg_sh.at[dstb[p].at[b]], dsem,
                                     add=True)
                    if b < G - 1:
                        pltpu.async_copy(x_hbm.at[srcb[p].at[b + 1]],
                                         rows[1 - b2], gsem[1 - b2])
                        pltpu.sync_copy(rows[b2], agg_sh.at[dstb[p].at[b]],
                                        add=True)
                    else:
                        pltpu.sync_copy(rows[b2], agg_sh.at[dstb[p].at[b]],
                                        add=True)

                        @pl.when(g < n_groups - 1)
                        def _():
                            pltpu.make_async_copy(
                                src_hbm.at[wid, pl.ds(0, G)], srcb[1 - p],
                                isem).wait()
                            pltpu.make_async_copy(
                                dst_hbm.at[wid, pl.ds(0, G)], dstb[1 - p],
                                isem).wait()
                            pltpu.async_copy(x_hbm.at[srcb[1 - p].at[0]],
                                             rows[1 - b2], gsem[1 - b2])
            return 0
        lax.fori_loop(0, n_groups // 2, gpair, 0)

        for _ in range(2 * G):
            pltpu.make_async_copy(ones_v, deg_sh.at[dstb[0].at[0]],
                                  dsem).wait()

        plsc.subcore_barrier()

        pltpu.sync_copy(agg_sh.at[pl.ds(r0, rows_per_tile)],
                        agg_out.at[c, pl.ds(r0, rows_per_tile)])
        pltpu.sync_copy(deg_sh.at[pl.ds(q0, deg_per_tile)],
                        deg_out.at[c, pl.ds(q0, deg_per_tile)])

    return body


def _tc_combine(x, a0, a1, deg2, w_self, w_neigh, b2, blk):
    n, d = x.shape

    def body(x_ref, a0_ref, a1_ref, deg_ref, ws_ref, wn_ref, b_ref, o_ref):
        agg = a0_ref[...] + a1_ref[...]
        deg = deg_ref[...]
        degsum = jnp.maximum(deg[:, 0] + deg[:, 1], 1.0)
        h_neigh = agg / degsum[:, None]
        o_ref[...] = (
            jnp.dot(x_ref[...], ws_ref[...], preferred_element_type=jnp.float32)
            + jnp.dot(h_neigh, wn_ref[...], preferred_element_type=jnp.float32)
            + b_ref[...]
        )

    grid = (n // blk,)
    return pl.pallas_call(
        body,
        grid=grid,
        in_specs=[
            pl.BlockSpec((blk, d), lambda i: (i, 0)),
            pl.BlockSpec((blk, d), lambda i: (i, 0)),
            pl.BlockSpec((blk, d), lambda i: (i, 0)),
            pl.BlockSpec((blk, NC), lambda i: (i, 0)),
            pl.BlockSpec((d, d), lambda i: (0, 0)),
            pl.BlockSpec((d, d), lambda i: (0, 0)),
            pl.BlockSpec((1, d), lambda i: (0, 0)),
        ],
        out_specs=pl.BlockSpec((blk, d), lambda i: (i, 0)),
        out_shape=jax.ShapeDtypeStruct((n, d), jnp.float32),
    )(x, a0, a1, deg2, w_self, w_neigh, b2)


def kernel(inputs, edge_index, layer_id, n_layers, W_self, W_neigh, b):
    n, d = inputs.shape
    e = edge_index.shape[1]

    per_step = NW * CHUNK
    n_chunks = -(-e // (per_step * G)) * G
    e_pad = n_chunks * per_step
    n_pad = -(-(n + 1) // 128) * 128

    src = edge_index[0]
    dst = edge_index[1]
    pad = e_pad - e
    if pad:
        pad_ids = jnp.arange(pad, dtype=jnp.int32)
        src = jnp.concatenate([src, pad_ids % n])
        dst = jnp.concatenate([dst, n + pad_ids % (n_pad - n)])
    src_t = src.reshape(NW, n_chunks, CHUNK)
    dst_t = dst.reshape(NW, n_chunks, CHUNK)

    agg_parts, deg_parts = _sc_aggregate(n_pad, d, n_chunks)(src_t, dst_t,
                                                             inputs)

    a0 = agg_parts[0]
    a1 = agg_parts[1]
    deg2 = deg_parts.reshape(NC, -1)[:, :n].T
    b2 = b.reshape(1, d)
    return _tc_combine(inputs, a0, a1, deg2, W_self, W_neigh, b2, blk=1000)


# speedup vs baseline: 3.8388x; 1.0330x over previous
import functools

import jax
import jax.numpy as jnp
from jax import lax
from jax.experimental import pallas as pl
from jax.experimental.pallas import tpu as pltpu
from jax.experimental.pallas import tpu_sc as plsc

NC = 2
NS = 16
NW = NC * NS
L = 16
CHUNK = 128
G = 8


def _sc_aggregate(n_pad, d, n_chunks):
    rows_per_tile = n_pad // NS
    deg_pad = -(-n_pad // (NS * L)) * (NS * L)
    deg_per_tile = deg_pad // NS
    n_groups = n_chunks // G

    mesh = plsc.VectorSubcoreMesh(core_axis_name="c", subcore_axis_name="s",
                                  num_cores=NC, num_subcores=NS)

    @functools.partial(
        pl.kernel,
        out_type=(
            jax.ShapeDtypeStruct((NC, n_pad, d), jnp.float32),
            jax.ShapeDtypeStruct((NC, deg_pad), jnp.float32),
        ),
        mesh=mesh,
        scratch_types=(
            [pltpu.VMEM((G, CHUNK), jnp.int32) for _ in range(2)],
            [pltpu.VMEM((G, CHUNK), jnp.int32) for _ in range(2)],
            [pltpu.VMEM((CHUNK, d), jnp.float32) for _ in range(2)],
            pltpu.VMEM((CHUNK,), jnp.float32),
            pltpu.VMEM((deg_per_tile,), jnp.float32),
            pltpu.VMEM_SHARED((n_pad, d), jnp.float32),
            pltpu.VMEM_SHARED((deg_pad,), jnp.float32),
            [pltpu.SemaphoreType.DMA for _ in range(2)],
            [pltpu.SemaphoreType.DMA for _ in range(2)],
            pltpu.SemaphoreType.DMA,
            pltpu.SemaphoreType.DMA,
            pltpu.SemaphoreType.DMA,
        ),
    )
    def body(src_hbm, dst_hbm, x_hbm, agg_out, deg_out,
             srcb, dstb, rows, ones_v, zdeg_v, agg_sh, deg_sh,
             gsem, ssem, dsem, isem, zsem):
        c = lax.axis_index("c")
        s = lax.axis_index("s")
        wid = c * NS + s
        r0 = s * rows_per_tile
        q0 = s * deg_per_tile
        zeros16 = jnp.zeros((L,), jnp.float32)
        ones16 = jnp.ones((L,), jnp.float32)

        def zero_row(i, _):
            for k in range(d // L):
                rows[1][i, pl.ds(k * L, L)] = zeros16
            return 0
        lax.fori_loop(0, CHUNK, zero_row, 0)
        for k in range(CHUNK // L):
            ones_v[pl.ds(k * L, L)] = ones16

        def zero_deg(i, _):
            zdeg_v[pl.ds(i * L, L)] = zeros16
            return 0
        lax.fori_loop(0, deg_per_tile // L, zero_deg, 0)

        descs = []
        nfull = rows_per_tile // CHUNK
        rem = rows_per_tile - nfull * CHUNK
        for k in range(nfull):
            descs.append(pltpu.async_copy(
                rows[1], agg_sh.at[pl.ds(r0 + k * CHUNK, CHUNK)], zsem))
        if rem:
            descs.append(pltpu.async_copy(
                rows[1].at[pl.ds(0, rem)],
                agg_sh.at[pl.ds(r0 + nfull * CHUNK, rem)], zsem))
        descs.append(pltpu.async_copy(
            zdeg_v, deg_sh.at[pl.ds(q0, deg_per_tile)], zsem))

        pltpu.sync_copy(src_hbm.at[wid, pl.ds(0, G)], srcb[0])
        pltpu.sync_copy(dst_hbm.at[wid, pl.ds(0, G)], dstb[0])
        pltpu.async_copy(x_hbm.at[srcb[0].at[0]], rows[0], gsem[0])

        for desc in descs:
            desc.wait()
        plsc.subcore_barrier()

        def gpair(g2, _):
            for p in (0, 1):
                g = g2 * 2 + p

                if p == 0:
                    @pl.when(g2 > 0)
                    def _():
                        pltpu.make_async_copy(
                            rows[1], agg_sh.at[dstb[1].at[0]], ssem[1]).wait()
                        for _b in range(G):
                            pltpu.make_async_copy(
                                ones_v, deg_sh.at[dstb[1].at[0]], dsem).wait()

                    @pl.when(g < n_groups - 1)
                    def _():
                        pltpu.async_copy(
                            src_hbm.at[wid, pl.ds((g + 1) * G, G)],
                            srcb[1], isem)
                        pltpu.async_copy(
                            dst_hbm.at[wid, pl.ds((g + 1) * G, G)],
                            dstb[1], isem)
                else:
                    @pl.when(g < n_groups - 1)
                    def _():
                        pltpu.make_async_copy(
                            rows[1], agg_sh.at[dstb[0].at[0]], ssem[1]).wait()
                        for _b in range(G):
                            pltpu.make_async_copy(
                                ones_v, deg_sh.at[dstb[0].at[0]], dsem).wait()
                        pltpu.async_copy(
                            src_hbm.at[wid, pl.ds((g + 1) * G, G)],
                            srcb[0], isem)
                        pltpu.async_copy(
                            dst_hbm.at[wid, pl.ds((g + 1) * G, G)],
                            dstb[0], isem)

                for b in range(G):
                    b2 = b & 1
                    pltpu.make_async_copy(
                        x_hbm.at[srcb[p].at[b]], rows[b2], gsem[b2]).wait()
                    pltpu.async_copy(ones_v, deg_sh.at[dstb[p].at[b]], dsem,
                                     add=True)
                    pltpu.async_copy(rows[b2], agg_sh.at[dstb[p].at[b]],
                                     ssem[b2], add=True)

                    if b > 0:
                        pltpu.make_async_copy(
                            rows[1 - b2], agg_sh.at[dstb[p].at[b]],
                            ssem[1 - b2]).wait()

                    if b < G - 1:
                        pltpu.async_copy(x_hbm.at[srcb[p].at[b + 1]],
                                         rows[1 - b2], gsem[1 - b2])
                    else:
                        @pl.when(g < n_groups - 1)
                        def _():
                            pltpu.make_async_copy(
                                src_hbm.at[wid, pl.ds(0, G)], srcb[1 - p],
                                isem).wait()
                            pltpu.make_async_copy(
                                dst_hbm.at[wid, pl.ds(0, G)], dstb[1 - p],
                                isem).wait()
                            pltpu.async_copy(x_hbm.at[srcb[1 - p].at[0]],
                                             rows[1 - b2], gsem[1 - b2])
            return 0
        lax.fori_loop(0, n_groups // 2, gpair, 0)

        for _ in range(2):
            pltpu.make_async_copy(rows[1], agg_sh.at[dstb[0].at[0]],
                                  ssem[1]).wait()
        for _ in range(2 * G):
            pltpu.make_async_copy(ones_v, deg_sh.at[dstb[0].at[0]],
                                  dsem).wait()

        plsc.subcore_barrier()

        pltpu.sync_copy(agg_sh.at[pl.ds(r0, rows_per_tile)],
                        agg_out.at[c, pl.ds(r0, rows_per_tile)])
        pltpu.sync_copy(deg_sh.at[pl.ds(q0, deg_per_tile)],
                        deg_out.at[c, pl.ds(q0, deg_per_tile)])

    return body


def _tc_combine(x, a0, a1, deg2, w_self, w_neigh, b2, blk):
    n, d = x.shape

    def body(x_ref, a0_ref, a1_ref, deg_ref, ws_ref, wn_ref, b_ref, o_ref):
        agg = a0_ref[...] + a1_ref[...]
        deg = deg_ref[...]
        degsum = jnp.maximum(deg[:, 0] + deg[:, 1], 1.0)
        h_neigh = agg / degsum[:, None]
        o_ref[...] = (
            jnp.dot(x_ref[...], ws_ref[...], preferred_element_type=jnp.float32)
            + jnp.dot(h_neigh, wn_ref[...], preferred_element_type=jnp.float32)
            + b_ref[...]
        )

    grid = (n // blk,)
    return pl.pallas_call(
        body,
        grid=grid,
        in_specs=[
            pl.BlockSpec((blk, d), lambda i: (i, 0)),
            pl.BlockSpec((blk, d), lambda i: (i, 0)),
            pl.BlockSpec((blk, d), lambda i: (i, 0)),
            pl.BlockSpec((blk, NC), lambda i: (i, 0)),
            pl.BlockSpec((d, d), lambda i: (0, 0)),
            pl.BlockSpec((d, d), lambda i: (0, 0)),
            pl.BlockSpec((1, d), lambda i: (0, 0)),
        ],
        out_specs=pl.BlockSpec((blk, d), lambda i: (i, 0)),
        out_shape=jax.ShapeDtypeStruct((n, d), jnp.float32),
    )(x, a0, a1, deg2, w_self, w_neigh, b2)


def kernel(inputs, edge_index, layer_id, n_layers, W_self, W_neigh, b):
    n, d = inputs.shape
    e = edge_index.shape[1]

    per_step = NW * CHUNK
    n_chunks = -(-e // (per_step * G)) * G
    e_pad = n_chunks * per_step
    n_pad = -(-(n + 1) // 128) * 128

    src = edge_index[0]
    dst = edge_index[1]
    pad = e_pad - e
    if pad:
        pad_ids = jnp.arange(pad, dtype=jnp.int32)
        src = jnp.concatenate([src, pad_ids % n])
        dst = jnp.concatenate([dst, n + pad_ids % (n_pad - n)])
    src_t = src.reshape(NW, n_chunks, CHUNK)
    dst_t = dst.reshape(NW, n_chunks, CHUNK)

    agg_parts, deg_parts = _sc_aggregate(n_pad, d, n_chunks)(src_t, dst_t,
                                                             inputs)

    a0 = agg_parts[0]
    a1 = agg_parts[1]
    deg2 = deg_parts.reshape(NC, -1)[:, :n].T
    b2 = b.reshape(1, d)
    return _tc_combine(inputs, a0, a1, deg2, W_self, W_neigh, b2, blk=1000)


# speedup vs baseline: 4.0191x; 1.0470x over previous
import functools

import jax
import jax.numpy as jnp
from jax import lax
from jax.experimental import pallas as pl
from jax.experimental.pallas import tpu as pltpu
from jax.experimental.pallas import tpu_sc as plsc

NC = 2
NS = 16
NW = NC * NS
L = 16
CHUNK = 128
G = 8


def _sc_aggregate(n_pad, d, n_chunks):
    rows_per_tile = n_pad // NS
    deg_pad = -(-n_pad // (NS * L)) * (NS * L)
    deg_per_tile = deg_pad // NS
    n_groups = n_chunks // G

    mesh = plsc.VectorSubcoreMesh(core_axis_name="c", subcore_axis_name="s",
                                  num_cores=NC, num_subcores=NS)

    @functools.partial(
        pl.kernel,
        out_type=(
            jax.ShapeDtypeStruct((NC, n_pad, d), jnp.float32),
            jax.ShapeDtypeStruct((NC, deg_pad), jnp.float32),
        ),
        mesh=mesh,
        scratch_types=(
            [pltpu.VMEM((G, CHUNK), jnp.int32) for _ in range(2)],
            [pltpu.VMEM((G, CHUNK), jnp.int32) for _ in range(2)],
            [pltpu.VMEM((CHUNK, d), jnp.float32) for _ in range(2)],
            pltpu.VMEM((CHUNK,), jnp.float32),
            pltpu.VMEM((deg_per_tile,), jnp.float32),
            pltpu.VMEM_SHARED((n_pad, d), jnp.float32),
            pltpu.VMEM_SHARED((deg_pad,), jnp.float32),
            [pltpu.SemaphoreType.DMA for _ in range(2)],
            [pltpu.SemaphoreType.DMA for _ in range(2)],
            pltpu.SemaphoreType.DMA,
            pltpu.SemaphoreType.DMA,
            pltpu.SemaphoreType.DMA,
        ),
    )
    def body(src_hbm, dst_hbm, x_hbm, agg_out, deg_out,
             srcb, dstb, rows, ones_v, zdeg_v, agg_sh, deg_sh,
             gsem, ssem, dsem, isem, zsem):
        c = lax.axis_index("c")
        s = lax.axis_index("s")
        wid = c * NS + s
        r0 = s * rows_per_tile
        q0 = s * deg_per_tile
        zeros16 = jnp.zeros((L,), jnp.float32)
        ones16 = jnp.ones((L,), jnp.float32)

        pltpu.sync_copy(src_hbm.at[wid, pl.ds(0, G)], srcb[0])
        pltpu.sync_copy(dst_hbm.at[wid, pl.ds(0, G)], dstb[0])
        pltpu.async_copy(x_hbm.at[srcb[0].at[0]], rows[0], gsem[0])

        def zero_row(i, _):
            for k in range(d // L):
                rows[1][i, pl.ds(k * L, L)] = zeros16
            return 0
        lax.fori_loop(0, CHUNK, zero_row, 0)
        for k in range(CHUNK // L):
            ones_v[pl.ds(k * L, L)] = ones16

        def zero_deg(i, _):
            zdeg_v[pl.ds(i * L, L)] = zeros16
            return 0
        lax.fori_loop(0, deg_per_tile // L, zero_deg, 0)

        descs = []
        nfull = rows_per_tile // CHUNK
        rem = rows_per_tile - nfull * CHUNK
        for k in range(nfull):
            descs.append(pltpu.async_copy(
                rows[1], agg_sh.at[pl.ds(r0 + k * CHUNK, CHUNK)], zsem))
        if rem:
            descs.append(pltpu.async_copy(
                rows[1].at[pl.ds(0, rem)],
                agg_sh.at[pl.ds(r0 + nfull * CHUNK, rem)], zsem))
        descs.append(pltpu.async_copy(
            zdeg_v, deg_sh.at[pl.ds(q0, deg_per_tile)], zsem))
        for desc in descs:
            desc.wait()
        plsc.subcore_barrier()

        def gpair(g2, _):
            for p in (0, 1):
                g = g2 * 2 + p

                if p == 0:
                    @pl.when(g2 > 0)
                    def _():
                        pltpu.make_async_copy(
                            rows[1], agg_sh.at[dstb[1].at[0]], ssem[1]).wait()
                        for _b in range(G):
                            pltpu.make_async_copy(
                                ones_v, deg_sh.at[dstb[1].at[0]], dsem).wait()

                    @pl.when(g < n_groups - 1)
                    def _():
                        pltpu.async_copy(
                            src_hbm.at[wid, pl.ds((g + 1) * G, G)],
                            srcb[1], isem)
                        pltpu.async_copy(
                            dst_hbm.at[wid, pl.ds((g + 1) * G, G)],
                            dstb[1], isem)
                else:
                    @pl.when(g < n_groups - 1)
                    def _():
                        pltpu.make_async_copy(
                            rows[1], agg_sh.at[dstb[0].at[0]], ssem[1]).wait()
                        for _b in range(G):
                            pltpu.make_async_copy(
                                ones_v, deg_sh.at[dstb[0].at[0]], dsem).wait()
                        pltpu.async_copy(
                            src_hbm.at[wid, pl.ds((g + 1) * G, G)],
                            srcb[0], isem)
                        pltpu.async_copy(
                            dst_hbm.at[wid, pl.ds((g + 1) * G, G)],
                            dstb[0], isem)

                for b in range(G):
                    b2 = b & 1
                    pltpu.make_async_copy(
                        x_hbm.at[srcb[p].at[b]], rows[b2], gsem[b2]).wait()
                    pltpu.async_copy(ones_v, deg_sh.at[dstb[p].at[b]], dsem,
                                     add=True)
                    pltpu.async_copy(rows[b2], agg_sh.at[dstb[p].at[b]],
                                     ssem[b2], add=True)

                    if b > 0:
                        pltpu.make_async_copy(
                            rows[1 - b2], agg_sh.at[dstb[p].at[b]],
                            ssem[1 - b2]).wait()

                    if b < G - 1:
                        pltpu.async_copy(x_hbm.at[srcb[p].at[b + 1]],
                                         rows[1 - b2], gsem[1 - b2])
                    else:
                        @pl.when(g < n_groups - 1)
                        def _():
                            pltpu.make_async_copy(
                                src_hbm.at[wid, pl.ds(0, G)], srcb[1 - p],
                                isem).wait()
                            pltpu.make_async_copy(
                                dst_hbm.at[wid, pl.ds(0, G)], dstb[1 - p],
                                isem).wait()
                            pltpu.async_copy(x_hbm.at[srcb[1 - p].at[0]],
                                             rows[1 - b2], gsem[1 - b2])
            return 0
        lax.fori_loop(0, n_groups // 2, gpair, 0)

        for _ in range(2):
            pltpu.make_async_copy(rows[1], agg_sh.at[dstb[0].at[0]],
                                  ssem[1]).wait()
        for _ in range(2 * G):
            pltpu.make_async_copy(ones_v, deg_sh.at[dstb[0].at[0]],
                                  dsem).wait()

        plsc.subcore_barrier()

        pltpu.sync_copy(agg_sh.at[pl.ds(r0, rows_per_tile)],
                        agg_out.at[c, pl.ds(r0, rows_per_tile)])
        pltpu.sync_copy(deg_sh.at[pl.ds(q0, deg_per_tile)],
                        deg_out.at[c, pl.ds(q0, deg_per_tile)])

    return body


def _tc_combine(x, agg_parts, deg2, w_self, w_neigh, b2, blk):
    n, d = x.shape

    def body(x_ref, a0_ref, a1_ref, deg_ref, ws_ref, wn_ref, b_ref, o_ref):
        agg = a0_ref[0] + a1_ref[0]
        deg = deg_ref[...]
        degsum = jnp.maximum(deg[:, 0] + deg[:, 1], 1.0)
        h_neigh = agg / degsum[:, None]
        o_ref[...] = (
            jnp.dot(x_ref[...], ws_ref[...], preferred_element_type=jnp.float32)
            + jnp.dot(h_neigh, wn_ref[...], preferred_element_type=jnp.float32)
            + b_ref[...]
        )

    grid = (n // blk,)
    return pl.pallas_call(
        body,
        grid=grid,
        in_specs=[
            pl.BlockSpec((blk, d), lambda i: (i, 0)),
            pl.BlockSpec((1, blk, d), lambda i: (0, i, 0)),
            pl.BlockSpec((1, blk, d), lambda i: (1, i, 0)),
            pl.BlockSpec((blk, NC), lambda i: (i, 0)),
            pl.BlockSpec((d, d), lambda i: (0, 0)),
            pl.BlockSpec((d, d), lambda i: (0, 0)),
            pl.BlockSpec((1, d), lambda i: (0, 0)),
        ],
        out_specs=pl.BlockSpec((blk, d), lambda i: (i, 0)),
        out_shape=jax.ShapeDtypeStruct((n, d), jnp.float32),
    )(x, agg_parts, agg_parts, deg2, w_self, w_neigh, b2)


def kernel(inputs, edge_index, layer_id, n_layers, W_self, W_neigh, b):
    n, d = inputs.shape
    e = edge_index.shape[1]

    per_step = NW * CHUNK
    n_chunks = -(-e // (per_step * G)) * G
    e_pad = n_chunks * per_step
    n_pad = -(-(n + 1) // 128) * 128

    src = edge_index[0]
    dst = edge_index[1]
    pad = e_pad - e
    if pad:
        pad_ids = jnp.arange(pad, dtype=jnp.int32)
        src = jnp.concatenate([src, pad_ids % n])
        dst = jnp.concatenate([dst, n + pad_ids % (n_pad - n)])
    src_t = src.reshape(NW, n_chunks, CHUNK)
    dst_t = dst.reshape(NW, n_chunks, CHUNK)

    agg_parts, deg_parts = _sc_aggregate(n_pad, d, n_chunks)(src_t, dst_t,
                                                             inputs)

    deg2 = deg_parts.reshape(NC, -1)[:, :n].T
    b2 = b.reshape(1, d)
    return _tc_combine(inputs, agg_parts, deg2, W_self, W_neigh, b2, blk=2000)


# speedup vs baseline: 4.0480x; 1.0072x over previous
import functools

import jax
import jax.numpy as jnp
from jax import lax
from jax.experimental import pallas as pl
from jax.experimental.pallas import tpu as pltpu
from jax.experimental.pallas import tpu_sc as plsc

NC = 2
NS = 16
NW = NC * NS
L = 16
CHUNK = 128
G = 8


def _sc_aggregate(n_pad, d, n_chunks):
    rows_per_tile = n_pad // NS
    deg_pad = -(-n_pad // (NS * L)) * (NS * L)
    deg_per_tile = deg_pad // NS
    n_groups = n_chunks // G

    mesh = plsc.VectorSubcoreMesh(core_axis_name="c", subcore_axis_name="s",
                                  num_cores=NC, num_subcores=NS)

    @functools.partial(
        pl.kernel,
        out_type=(
            jax.ShapeDtypeStruct((NC, n_pad, d), jnp.float32),
            jax.ShapeDtypeStruct((NC, deg_pad), jnp.float32),
        ),
        mesh=mesh,
        scratch_types=(
            [pltpu.VMEM((G, CHUNK), jnp.int32) for _ in range(2)],
            [pltpu.VMEM((G, CHUNK), jnp.int32) for _ in range(2)],
            [pltpu.VMEM((CHUNK, d), jnp.float32) for _ in range(2)],
            pltpu.VMEM((CHUNK,), jnp.float32),
            pltpu.VMEM((deg_per_tile,), jnp.float32),
            pltpu.VMEM_SHARED((n_pad, d), jnp.float32),
            pltpu.VMEM_SHARED((deg_pad,), jnp.float32),
            [pltpu.SemaphoreType.DMA for _ in range(2)],
            [pltpu.SemaphoreType.DMA for _ in range(2)],
            pltpu.SemaphoreType.DMA,
            pltpu.SemaphoreType.DMA,
            pltpu.SemaphoreType.DMA,
        ),
    )
    def body(src_hbm, dst_hbm, x_hbm, agg_out, deg_out,
             srcb, dstb, rows, ones_v, zdeg_v, agg_sh, deg_sh,
             gsem, ssem, dsem, isem, zsem):
        c = lax.axis_index("c")
        s = lax.axis_index("s")
        wid = c * NS + s
        r0 = s * rows_per_tile
        q0 = s * deg_per_tile
        zeros16 = jnp.zeros((L,), jnp.float32)
        ones16 = jnp.ones((L,), jnp.float32)

        pltpu.sync_copy(src_hbm.at[wid, pl.ds(0, G)], srcb[0])
        pltpu.sync_copy(dst_hbm.at[wid, pl.ds(0, G)], dstb[0])
        pltpu.async_copy(x_hbm.at[srcb[0].at[0]], rows[0], gsem[0])

        def zero_row(i, _):
            for k in range(d // L):
                rows[1][i, pl.ds(k * L, L)] = zeros16
            return 0
        lax.fori_loop(0, CHUNK, zero_row, 0)
        for k in range(CHUNK // L):
            ones_v[pl.ds(k * L, L)] = ones16

        def zero_deg(i, _):
            zdeg_v[pl.ds(i * L, L)] = zeros16
            return 0
        lax.fori_loop(0, deg_per_tile // L, zero_deg, 0)

        descs = []
        nfull = rows_per_tile // CHUNK
        rem = rows_per_tile - nfull * CHUNK
        for k in range(nfull):
            descs.append(pltpu.async_copy(
                rows[1], agg_sh.at[pl.ds(r0 + k * CHUNK, CHUNK)], zsem))
        if rem:
            descs.append(pltpu.async_copy(
                rows[1].at[pl.ds(0, rem)],
                agg_sh.at[pl.ds(r0 + nfull * CHUNK, rem)], zsem))
        descs.append(pltpu.async_copy(
            zdeg_v, deg_sh.at[pl.ds(q0, deg_per_tile)], zsem))
        for desc in descs:
            desc.wait()
        plsc.subcore_barrier()

        def gpair(g2, _):
            for p in (0, 1):
                g = g2 * 2 + p

                if p == 0:
                    @pl.when(g2 > 0)
                    def _():
                        pltpu.make_async_copy(
                            rows[1], agg_sh.at[dstb[1].at[0]], ssem[1]).wait()
                        for _b in range(G):
                            pltpu.make_async_copy(
                                ones_v, deg_sh.at[dstb[1].at[0]], dsem).wait()

                    @pl.when(g < n_groups - 1)
                    def _():
                        pltpu.async_copy(
                            src_hbm.at[wid, pl.ds((g + 1) * G, G)],
                            srcb[1], isem)
                        pltpu.async_copy(
                            dst_hbm.at[wid, pl.ds((g + 1) * G, G)],
                            dstb[1], isem)
                else:
                    @pl.when(g < n_groups - 1)
                    def _():
                        pltpu.make_async_copy(
                            rows[1], agg_sh.at[dstb[0].at[0]], ssem[1]).wait()
                        for _b in range(G):
                            pltpu.make_async_copy(
                                ones_v, deg_sh.at[dstb[0].at[0]], dsem).wait()
                        pltpu.async_copy(
                            src_hbm.at[wid, pl.ds((g + 1) * G, G)],
                            srcb[0], isem)
                        pltpu.async_copy(
                            dst_hbm.at[wid, pl.ds((g + 1) * G, G)],
                            dstb[0], isem)

                for b in range(G):
                    b2 = b & 1
                    pltpu.make_async_copy(
                        x_hbm.at[srcb[p].at[b]], rows[b2], gsem[b2]).wait()
                    pltpu.async_copy(ones_v, deg_sh.at[dstb[p].at[b]], dsem,
                                     add=True)
                    pltpu.async_copy(rows[b2], agg_sh.at[dstb[p].at[b]],
                                     ssem[b2], add=True)

                    if b > 0:
                        pltpu.make_async_copy(
                            rows[1 - b2], agg_sh.at[dstb[p].at[b]],
                            ssem[1 - b2]).wait()

                    if b < G - 1:
                        pltpu.async_copy(x_hbm.at[srcb[p].at[b + 1]],
                                         rows[1 - b2], gsem[1 - b2])
                    else:
                        @pl.when(g < n_groups - 1)
                        def _():
                            pltpu.make_async_copy(
                                src_hbm.at[wid, pl.ds(0, G)], srcb[1 - p],
                                isem).wait()
                            pltpu.make_async_copy(
                                dst_hbm.at[wid, pl.ds(0, G)], dstb[1 - p],
                                isem).wait()
                            pltpu.async_copy(x_hbm.at[srcb[1 - p].at[0]],
                                             rows[1 - b2], gsem[1 - b2])
            return 0
        lax.fori_loop(0, n_groups // 2, gpair, 0)

        for _ in range(2):
            pltpu.make_async_copy(rows[1], agg_sh.at[dstb[0].at[0]],
                                  ssem[1]).wait()
        for _ in range(2 * G):
            pltpu.make_async_copy(ones_v, deg_sh.at[dstb[0].at[0]],
                                  dsem).wait()

        plsc.subcore_barrier()

        pltpu.sync_copy(agg_sh.at[pl.ds(r0, rows_per_tile)],
                        agg_out.at[c, pl.ds(r0, rows_per_tile)])
        pltpu.sync_copy(deg_sh.at[pl.ds(q0, deg_per_tile)],
                        deg_out.at[c, pl.ds(q0, deg_per_tile)])

    return body


def _tc_combine(x, agg_parts, deg2, w_self, w_neigh, b2, blk):
    n, d = x.shape

    def body(x_ref, a0_ref, a1_ref, deg_ref, ws_ref, wn_ref, b_ref, o_ref):
        agg = a0_ref[0] + a1_ref[0]
        deg = deg_ref[...]
        degsum = jnp.maximum(deg[:, 0] + deg[:, 1], 1.0)
        h_neigh = agg / degsum[:, None]
        o_ref[...] = (
            jnp.dot(x_ref[...], ws_ref[...], preferred_element_type=jnp.float32)
            + jnp.dot(h_neigh, wn_ref[...], preferred_element_type=jnp.float32)
            + b_ref[...]
        )

    grid = (n // blk,)
    return pl.pallas_call(
        body,
        grid=grid,
        in_specs=[
            pl.BlockSpec((blk, d), lambda i: (i, 0)),
            pl.BlockSpec((1, blk, d), lambda i: (0, i, 0)),
            pl.BlockSpec((1, blk, d), lambda i: (1, i, 0)),
            pl.BlockSpec((blk, NC), lambda i: (i, 0)),
            pl.BlockSpec((d, d), lambda i: (0, 0)),
            pl.BlockSpec((d, d), lambda i: (0, 0)),
            pl.BlockSpec((1, d), lambda i: (0, 0)),
        ],
        out_specs=pl.BlockSpec((blk, d), lambda i: (i, 0)),
        out_shape=jax.ShapeDtypeStruct((n, d), jnp.float32),
    )(x, agg_parts, agg_parts, deg2, w_self, w_neigh, b2)


def kernel(inputs, edge_index, layer_id, n_layers, W_self, W_neigh, b):
    n, d = inputs.shape
    e = edge_index.shape[1]

    per_step = NW * CHUNK
    n_chunks = -(-e // (per_step * G)) * G
    e_pad = n_chunks * per_step
    n_pad = -(-(n + 1) // 128) * 128

    src = edge_index[0]
    dst = edge_index[1]
    pad = e_pad - e
    if pad:
        pad_ids = jnp.arange(pad, dtype=jnp.int32)
        src = jnp.concatenate([src, pad_ids % n])
        dst = jnp.concatenate([dst, n + pad_ids % (n_pad - n)])
    src_t = src.reshape(NW, n_chunks, CHUNK)
    dst_t = dst.reshape(NW, n_chunks, CHUNK)

    agg_parts, deg_parts = _sc_aggregate(n_pad, d, n_chunks)(src_t, dst_t,
                                                             inputs)

    deg2 = deg_parts.reshape(NC, -1)[:, :n].T
    b2 = b.reshape(1, d)
    return _tc_combine(inputs, agg_parts, deg2, W_self, W_neigh, b2, blk=5000)
